# Initial kernel scaffold; baseline (speedup 1.0000x reference)
import jax
import jax.numpy as jnp
from jax.experimental import pallas as pl


def kernel(x, edge_index, edge_attr, W1, b1, gamma, beta, W2, b2):
    raise NotImplementedError("write your pallas kernel here")


# speedup vs baseline: 1.5092x; 1.5092x over previous
import functools

import jax
import jax.numpy as jnp
from jax import lax
from jax.experimental import pallas as pl
from jax.experimental.pallas import tpu as pltpu
from jax.experimental.pallas import tpu_sc as plsc

N = 10000
E = 160000
DIM = 256
HALF = 128
NC = 2
NS = 16
CH = 128
NCHUNKS = E // CH
WB = 80

_sc_mesh = plsc.VectorSubcoreMesh(core_axis_name="c", subcore_axis_name="s")


def _make_edge_kernel():
    @functools.partial(
        pl.kernel,
        out_type=jax.ShapeDtypeStruct((NC, N, HALF), jnp.float32),
        mesh=_sc_mesh,
        scratch_types=[
            pltpu.VMEM((CH,), jnp.int32),
            pltpu.VMEM((CH,), jnp.int32),
            pltpu.VMEM((CH,), jnp.int32),
            pltpu.VMEM((CH,), jnp.int32),
            pltpu.VMEM((CH, HALF), jnp.float32),
            pltpu.VMEM((CH, HALF), jnp.float32),
            pltpu.VMEM_SHARED((N, HALF), jnp.float32),
            pltpu.SemaphoreType.DMA,
            pltpu.SemaphoreType.DMA,
        ],
    )
    def edge_kernel(h2, srcg, dstg, ea2, out,
                    src_v, dst_v, idx_v, eidx_v, msg_v, ea_v, agg_sh,
                    sem_h, sem_e):
        c = lax.axis_index("c")
        s = lax.axis_index("s")

        def zrow(r, carry):
            for k in range(HALF // 16):
                msg_v[r, pl.ds(k * 16, 16)] = jnp.zeros((16,), jnp.float32)
            return carry
        lax.fori_loop(0, CH, zrow, 0, unroll=2)
        nblk = jnp.where(s < NS - 1, 8, 5)

        def zblk(t, carry):
            r0 = s * (8 * WB) + t * WB
            pltpu.sync_copy(msg_v.at[pl.ds(0, WB)], agg_sh.at[pl.ds(r0, WB)])
            return carry
        lax.fori_loop(0, nblk, zblk, 0)
        plsc.subcore_barrier()

        extra = NCHUNKS - (NCHUNKS // NS) * NS
        nchunks = jnp.where(s < extra, NCHUNKS // NS + 1, NCHUNKS // NS)

        def body(t, carry):
            base = (s + t * NS) * CH
            pltpu.sync_copy(srcg.at[pl.ds(base, CH)], src_v)
            pltpu.sync_copy(dstg.at[pl.ds(base, CH)], dst_v)
            for k in range(CH // 16):
                sl = pl.ds(k * 16, 16)
                idx_v[sl] = src_v[sl] * 2 + c
                eidx_v[sl] = (base + k * 16 + lax.iota(jnp.int32, 16)) * 2 + c
            cp_e = pltpu.async_copy(ea2.at[eidx_v], ea_v, sem_e)
            pltpu.async_copy(h2.at[idx_v], msg_v, sem_h).wait()
            cp_e.wait()

            def crow(r, carry2):
                for k in range(HALF // 16):
                    sl = pl.ds(k * 16, 16)
                    msg_v[r, sl] = jnp.maximum(msg_v[r, sl] + ea_v[r, sl], 0.0)
                return carry2
            lax.fori_loop(0, CH, crow, 0, unroll=2)

            pltpu.sync_copy(msg_v, agg_sh.at[dst_v], add=True)
            return carry
        lax.fori_loop(0, nchunks, body, 0)

        plsc.subcore_barrier()

        def wblk(t, carry):
            r0 = s * (8 * WB) + t * WB
            pltpu.sync_copy(agg_sh.at[pl.ds(r0, WB)], out.at[c, pl.ds(r0, WB)])
            return carry
        lax.fori_loop(0, nblk, wblk, 0)

    return edge_kernel


_edge_kernel = _make_edge_kernel()


BR = 1000
G = N // BR


def _mlp1_body(h_ref, a0_ref, a1_ref, w1_ref, b1_ref, z_ref, s_ref):
    y = h_ref[...] + jnp.concatenate([a0_ref[...], a1_ref[...]], axis=1)
    z = lax.dot_general(y, w1_ref[...], (((1,), (1,)), ((), ())),
                        preferred_element_type=jnp.float32) + b1_ref[...]
    z_ref[...] = z
    s_ref[0, 0, :] = jnp.sum(z, axis=0)
    s_ref[0, 1, :] = jnp.sum(z * z, axis=0)


def _mlp1(h, a0, a1, w1, b1):
    return pl.pallas_call(
        _mlp1_body,
        grid=(G,),
        in_specs=[
            pl.BlockSpec((BR, DIM), lambda b: (b, 0)),
            pl.BlockSpec((BR, HALF), lambda b: (b, 0)),
            pl.BlockSpec((BR, HALF), lambda b: (b, 0)),
            pl.BlockSpec((DIM, DIM), lambda b: (0, 0)),
            pl.BlockSpec((1, DIM), lambda b: (0, 0)),
        ],
        out_specs=[
            pl.BlockSpec((BR, DIM), lambda b: (b, 0)),
            pl.BlockSpec((1, 2, DIM), lambda b: (b, 0, 0)),
        ],
        out_shape=[
            jax.ShapeDtypeStruct((N, DIM), jnp.float32),
            jax.ShapeDtypeStruct((G, 2, DIM), jnp.float32),
        ],
    )(h, a0, a1, w1, b1)


def _mlp2_body(z_ref, s_ref, g_ref, be_ref, w2_ref, b2_ref, o_ref):
    srt = s_ref[...]
    mu = jnp.sum(srt[:, 0, :], axis=0) * (1.0 / N)
    msq = jnp.sum(srt[:, 1, :], axis=0) * (1.0 / N)
    var = msq - mu * mu
    inv = lax.rsqrt(var + 1e-5)
    u = (z_ref[...] - mu) * (inv * g_ref[0]) + be_ref[0]
    u = u * jax.nn.sigmoid(u)
    v = lax.dot_general(u, w2_ref[...], (((1,), (1,)), ((), ())),
                        preferred_element_type=jnp.float32) + b2_ref[...]
    o_ref[...] = v * jax.nn.sigmoid(v)


def _mlp2(z, sums, gamma, beta, w2, b2):
    return pl.pallas_call(
        _mlp2_body,
        grid=(G,),
        in_specs=[
            pl.BlockSpec((BR, DIM), lambda b: (b, 0)),
            pl.BlockSpec((G, 2, DIM), lambda b: (0, 0, 0)),
            pl.BlockSpec((1, DIM), lambda b: (0, 0)),
            pl.BlockSpec((1, DIM), lambda b: (0, 0)),
            pl.BlockSpec((DIM, DIM), lambda b: (0, 0)),
            pl.BlockSpec((1, DIM), lambda b: (0, 0)),
        ],
        out_specs=pl.BlockSpec((BR, DIM), lambda b: (b, 0)),
        out_shape=jax.ShapeDtypeStruct((N, DIM), jnp.float32),
    )(z, sums, gamma, beta, w2, b2)


def kernel(x, edge_index, edge_attr, W1, b1, gamma, beta, W2, b2):
    src = edge_index[0]
    dst = edge_index[1]
    ea2 = edge_attr.reshape(2 * E, HALF)
    h = x
    for i in range(3):
        h2 = h.reshape(2 * N, HALF)
        aggs = _edge_kernel(h2, src, dst, ea2)
        z, sums = _mlp1(h, aggs[0], aggs[1], W1[i],
                        b1[i].reshape(1, DIM))
        h = _mlp2(z, sums, gamma[i].reshape(1, DIM), beta[i].reshape(1, DIM),
                  W2[i], b2[i].reshape(1, DIM))
    return h


# speedup vs baseline: 1.9995x; 1.3249x over previous
import functools

import jax
import jax.numpy as jnp
from jax import lax
from jax.experimental import pallas as pl
from jax.experimental.pallas import tpu as pltpu
from jax.experimental.pallas import tpu_sc as plsc

N = 10000
E = 160000
DIM = 256
HALF = 128
NC = 2
NS = 16
CH = 96
NCHUNKS = -(-E // CH)
EPAD = NCHUNKS * CH
NA = N + 8
WB = 80

_sc_mesh = plsc.VectorSubcoreMesh(core_axis_name="c", subcore_axis_name="s")


GRP = 16
NT = 112


def _make_edge_kernel():
    @functools.partial(
        pl.kernel,
        out_type=jax.ShapeDtypeStruct((NC, N, HALF), jnp.float32),
        mesh=_sc_mesh,
        scratch_types=[
            pltpu.VMEM((CH,), jnp.int32),
            pltpu.VMEM((CH,), jnp.int32),
            pltpu.VMEM((CH,), jnp.int32),
            pltpu.VMEM((CH,), jnp.int32),
            pltpu.VMEM((CH,), jnp.int32),
            pltpu.VMEM((CH,), jnp.int32),
            pltpu.VMEM((CH,), jnp.int32),
            pltpu.VMEM((CH,), jnp.int32),
            pltpu.VMEM((CH,), jnp.int32),
            pltpu.VMEM((CH,), jnp.int32),
            pltpu.VMEM((CH,), jnp.int32),
            pltpu.VMEM((CH,), jnp.int32),
            pltpu.VMEM((CH, HALF), jnp.float32),
            pltpu.VMEM((CH, HALF), jnp.float32),
            pltpu.VMEM((CH, HALF), jnp.float32),
            pltpu.VMEM((CH, HALF), jnp.float32),
            pltpu.VMEM_SHARED((NA, HALF), jnp.float32),
            pltpu.SemaphoreType.DMA,
            pltpu.SemaphoreType.DMA,
            pltpu.SemaphoreType.DMA,
            pltpu.SemaphoreType.DMA,
            pltpu.SemaphoreType.DMA,
            pltpu.SemaphoreType.DMA,
        ],
    )
    def edge_kernel(h2, srcg, dstg, ea2, out,
                    sv0, sv1, sv2, sv3, dv0, dv1, dv2, dv3,
                    ix0, ix1, ex0, ex1, mg0, mg1, eb0, eb1, agg_sh,
                    isem0, isem1, gsem0, gsem1, ssem0, ssem1):
        c = lax.axis_index("c")
        s = lax.axis_index("s")
        src_bufs = (sv0, sv1, sv2, sv3)
        dst_bufs = (dv0, dv1, dv2, dv3)
        idxg_bufs = (ix0, ix1)
        eidx_bufs = (ex0, ex1)
        msg_bufs = (mg0, mg1)
        ea_bufs = (eb0, eb1)
        idx_sems = (isem0, isem1)
        gat_sems = (gsem0, gsem1)
        sct_sems = (ssem0, ssem1)

        def valid(u):
            return jnp.logical_and(u >= 0, s + u * NS < NCHUNKS)

        def fire_idx(u, b4):
            sem = idx_sems[b4 % 2]

            @pl.when(valid(u))
            def _():
                base = (s + u * NS) * CH
                pltpu.async_copy(srcg.at[pl.ds(base, CH)], src_bufs[b4], sem)
                pltpu.async_copy(dstg.at[pl.ds(base, CH)], dst_bufs[b4], sem)

        def wait_idx_compute_fire_gather(u, b4, b2):
            isem = idx_sems[b4 % 2]
            gsem = gat_sems[b2]
            src_v, dst_v = src_bufs[b4], dst_bufs[b4]
            idxg_v, eidx_v = idxg_bufs[b2], eidx_bufs[b2]

            @pl.when(valid(u))
            def _():
                base = (s + u * NS) * CH
                pltpu.make_async_copy(srcg.at[pl.ds(0, CH)], src_v,
                                      isem).wait()
                pltpu.make_async_copy(dstg.at[pl.ds(0, CH)], dst_v,
                                      isem).wait()
                for k in range(CH // 16):
                    sl = pl.ds(k * 16, 16)
                    idxg_v[sl] = src_v[sl] * 2 + c
                    e_ids = jnp.minimum(base + k * 16
                                        + lax.iota(jnp.int32, 16), E - 1)
                    eidx_v[sl] = e_ids * 2 + c
                pltpu.async_copy(h2.at[idxg_v], msg_bufs[b2], gsem)
                pltpu.async_copy(ea2.at[eidx_v], ea_bufs[b2], gsem)

        def drain_scatter(u, b2, b4):
            sem = sct_sems[b2]

            @pl.when(valid(u))
            def _():
                pltpu.make_async_copy(msg_bufs[b2],
                                      agg_sh.at[dst_bufs[b4]], sem).wait()

        def process(u, b2, b4):
            gsem = gat_sems[b2]
            ssem = sct_sems[b2]
            msg_v, ea_v = msg_bufs[b2], ea_bufs[b2]

            @pl.when(valid(u))
            def _():
                pltpu.make_async_copy(h2.at[idxg_bufs[b2]], msg_v,
                                      gsem).wait()
                pltpu.make_async_copy(ea2.at[eidx_bufs[b2]], ea_v,
                                      gsem).wait()

                def crow(r, carry2):
                    for k in range(HALF // 16):
                        sl = pl.ds(k * 16, 16)
                        msg_v[r, sl] = jnp.maximum(
                            msg_v[r, sl] + ea_v[r, sl], 0.0)
                    return carry2
                lax.fori_loop(0, CH, crow, 0, unroll=2)
                pltpu.async_copy(msg_v, agg_sh.at[dst_bufs[b4]], ssem)

        def zrow(r, carry):
            for k in range(HALF // 16):
                mg0[r, pl.ds(k * 16, 16)] = jnp.zeros((16,), jnp.float32)
            return carry
        lax.fori_loop(0, CH, zrow, 0, unroll=2)
        nblk = jnp.where(s < NS - 1, 8, 5)

        def zblk(t, carry):
            r0 = s * (8 * WB) + t * WB
            pltpu.sync_copy(mg0.at[pl.ds(0, WB)],
                            agg_sh.at[pl.ds(r0, WB)])
            return carry
        lax.fori_loop(0, nblk, zblk, 0)
        plsc.subcore_barrier()

        def group(t, carry):
            u0 = t * GRP

            idesc = []
            gdesc = []
            sdesc = []
            for i in range(GRP):
                base = (s + (u0 + i) * NS) * CH
                idesc.append((
                    pltpu.make_async_copy(srcg.at[pl.ds(base, CH)],
                                          src_bufs[i % 4], idx_sems[i % 2]),
                    pltpu.make_async_copy(dstg.at[pl.ds(base, CH)],
                                          dst_bufs[i % 4], idx_sems[i % 2]),
                ))
                gdesc.append((
                    pltpu.make_async_copy(h2.at[idxg_bufs[i % 2]],
                                          msg_bufs[i % 2], gat_sems[i % 2]),
                    pltpu.make_async_copy(ea2.at[eidx_bufs[i % 2]],
                                          ea_bufs[i % 2], gat_sems[i % 2]),
                ))
                sdesc.append(
                    pltpu.make_async_copy(msg_bufs[i % 2],
                                          agg_sh.at[dst_bufs[i % 4]],
                                          sct_sems[i % 2]))

            def fire_idx_i(i):
                @pl.when(valid(u0 + i))
                def _(i=i):
                    idesc[i][0].start()
                    idesc[i][1].start()

            def proc(i):
                @pl.when(valid(u0 + i))
                def _(i=i):
                    gdesc[i][0].wait()
                    gdesc[i][1].wait()
                    msg_v, ea_v = msg_bufs[i % 2], ea_bufs[i % 2]

                    def crow(r, carry2):
                        for k in range(HALF // 16):
                            sl = pl.ds(k * 16, 16)
                            msg_v[r, sl] = jnp.maximum(
                                msg_v[r, sl] + ea_v[r, sl], 0.0)
                        return carry2
                    lax.fori_loop(0, CH, crow, 0, unroll=2)
                    sdesc[i].start(add=True)

            fire_idx_i(0)
            fire_idx_i(1)
            for i in range(GRP):
                u = u0 + i

                @pl.when(valid(u))
                def _(i=i, u=u):
                    idesc[i][0].wait()
                    idesc[i][1].wait()
                    base = (s + u * NS) * CH
                    src_v = src_bufs[i % 4]
                    idxg_v, eidx_v = idxg_bufs[i % 2], eidx_bufs[i % 2]
                    for k in range(CH // 16):
                        sl = pl.ds(k * 16, 16)
                        idxg_v[sl] = src_v[sl] * 2 + c
                        e_ids = jnp.minimum(base + k * 16
                                            + lax.iota(jnp.int32, 16), E - 1)
                        eidx_v[sl] = e_ids * 2 + c
                if i >= 2:
                    @pl.when(valid(u0 + i - 2))
                    def _(i=i):
                        sdesc[i - 2].wait()

                @pl.when(valid(u))
                def _(i=i):
                    gdesc[i][0].start()
                    gdesc[i][1].start()
                if i + 2 < GRP:
                    fire_idx_i(i + 2)
                if i >= 1:
                    proc(i - 1)
            proc(GRP - 1)

            @pl.when(valid(u0 + GRP - 2))
            def _():
                sdesc[GRP - 2].wait()

            @pl.when(valid(u0 + GRP - 1))
            def _():
                sdesc[GRP - 1].wait()
            return carry
        lax.fori_loop(0, NT // GRP, group, 0)

        plsc.subcore_barrier()

        def wblk(t, carry):
            r0 = s * (8 * WB) + t * WB
            pltpu.sync_copy(agg_sh.at[pl.ds(r0, WB)], out.at[c, pl.ds(r0, WB)])
            return carry
        lax.fori_loop(0, nblk, wblk, 0)

    return edge_kernel


_edge_kernel = _make_edge_kernel()


BR = 1000
G = N // BR


def _mlp1_body(h_ref, a0_ref, a1_ref, w1_ref, b1_ref, z_ref, s_ref):
    y = h_ref[...] + jnp.concatenate([a0_ref[...], a1_ref[...]], axis=1)
    z = lax.dot_general(y, w1_ref[...], (((1,), (1,)), ((), ())),
                        preferred_element_type=jnp.float32) + b1_ref[...]
    z_ref[...] = z
    s_ref[0, 0, :] = jnp.sum(z, axis=0)
    s_ref[0, 1, :] = jnp.sum(z * z, axis=0)


def _mlp1(h, a0, a1, w1, b1):
    return pl.pallas_call(
        _mlp1_body,
        grid=(G,),
        in_specs=[
            pl.BlockSpec((BR, DIM), lambda b: (b, 0)),
            pl.BlockSpec((BR, HALF), lambda b: (b, 0)),
            pl.BlockSpec((BR, HALF), lambda b: (b, 0)),
            pl.BlockSpec((DIM, DIM), lambda b: (0, 0)),
            pl.BlockSpec((1, DIM), lambda b: (0, 0)),
        ],
        out_specs=[
            pl.BlockSpec((BR, DIM), lambda b: (b, 0)),
            pl.BlockSpec((1, 2, DIM), lambda b: (b, 0, 0)),
        ],
        out_shape=[
            jax.ShapeDtypeStruct((N, DIM), jnp.float32),
            jax.ShapeDtypeStruct((G, 2, DIM), jnp.float32),
        ],
    )(h, a0, a1, w1, b1)


def _mlp2_body(z_ref, s_ref, g_ref, be_ref, w2_ref, b2_ref, o_ref):
    srt = s_ref[...]
    mu = jnp.sum(srt[:, 0, :], axis=0) * (1.0 / N)
    msq = jnp.sum(srt[:, 1, :], axis=0) * (1.0 / N)
    var = msq - mu * mu
    inv = lax.rsqrt(var + 1e-5)
    u = (z_ref[...] - mu) * (inv * g_ref[0]) + be_ref[0]
    u = u * jax.nn.sigmoid(u)
    v = lax.dot_general(u, w2_ref[...], (((1,), (1,)), ((), ())),
                        preferred_element_type=jnp.float32) + b2_ref[...]
    o_ref[...] = v * jax.nn.sigmoid(v)


def _mlp2(z, sums, gamma, beta, w2, b2):
    return pl.pallas_call(
        _mlp2_body,
        grid=(G,),
        in_specs=[
            pl.BlockSpec((BR, DIM), lambda b: (b, 0)),
            pl.BlockSpec((G, 2, DIM), lambda b: (0, 0, 0)),
            pl.BlockSpec((1, DIM), lambda b: (0, 0)),
            pl.BlockSpec((1, DIM), lambda b: (0, 0)),
            pl.BlockSpec((DIM, DIM), lambda b: (0, 0)),
            pl.BlockSpec((1, DIM), lambda b: (0, 0)),
        ],
        out_specs=pl.BlockSpec((BR, DIM), lambda b: (b, 0)),
        out_shape=jax.ShapeDtypeStruct((N, DIM), jnp.float32),
    )(z, sums, gamma, beta, w2, b2)


def kernel(x, edge_index, edge_attr, W1, b1, gamma, beta, W2, b2):
    src = jnp.concatenate([edge_index[0],
                           jnp.zeros((EPAD - E,), jnp.int32)])
    dst = jnp.concatenate([edge_index[1],
                           jnp.full((EPAD - E,), N, jnp.int32)])
    ea2 = edge_attr.reshape(2 * E, HALF)
    h = x
    for i in range(3):
        h2 = h.reshape(2 * N, HALF)
        aggs = _edge_kernel(h2, src, dst, ea2)
        z, sums = _mlp1(h, aggs[0], aggs[1], W1[i],
                        b1[i].reshape(1, DIM))
        h = _mlp2(z, sums, gamma[i].reshape(1, DIM), beta[i].reshape(1, DIM),
                  W2[i], b2[i].reshape(1, DIM))
    return h


# speedup vs baseline: 2.0631x; 1.0318x over previous
import functools

import jax
import jax.numpy as jnp
from jax import lax
from jax.experimental import pallas as pl
from jax.experimental.pallas import tpu as pltpu
from jax.experimental.pallas import tpu_sc as plsc

N = 10000
E = 160000
DIM = 256
HALF = 128
NC = 2
NS = 16
CH = 96
NCHUNKS = -(-E // CH)
EPAD = NCHUNKS * CH
NA = N + 8
WB = 80

_sc_mesh = plsc.VectorSubcoreMesh(core_axis_name="c", subcore_axis_name="s")


GRP = 16
NT = 112


def _make_edge_kernel():
    @functools.partial(
        pl.kernel,
        out_type=jax.ShapeDtypeStruct((NC, N, HALF), jnp.float32),
        mesh=_sc_mesh,
        scratch_types=[
            pltpu.VMEM((CH,), jnp.int32),
            pltpu.VMEM((CH,), jnp.int32),
            pltpu.VMEM((CH,), jnp.int32),
            pltpu.VMEM((CH,), jnp.int32),
            pltpu.VMEM((CH,), jnp.int32),
            pltpu.VMEM((CH,), jnp.int32),
            pltpu.VMEM((CH,), jnp.int32),
            pltpu.VMEM((CH,), jnp.int32),
            pltpu.VMEM((CH,), jnp.int32),
            pltpu.VMEM((CH,), jnp.int32),
            pltpu.VMEM((CH, HALF), jnp.float32),
            pltpu.VMEM((CH, HALF), jnp.float32),
            pltpu.VMEM((CH, HALF), jnp.float32),
            pltpu.VMEM((CH, HALF), jnp.float32),
            pltpu.VMEM_SHARED((NA, HALF), jnp.float32),
            pltpu.SemaphoreType.DMA,
            pltpu.SemaphoreType.DMA,
            pltpu.SemaphoreType.DMA,
            pltpu.SemaphoreType.DMA,
            pltpu.SemaphoreType.DMA,
            pltpu.SemaphoreType.DMA,
        ],
    )
    def edge_kernel(h2, srcg, dstg, ea2, out,
                    sv0, sv1, sv2, sv3, dv0, dv1, dv2, dv3,
                    ix0, ix1, mg0, mg1, eb0, eb1, agg_sh,
                    isem0, isem1, gsem0, gsem1, ssem0, ssem1):
        c = lax.axis_index("c")
        s = lax.axis_index("s")
        src_bufs = (sv0, sv1, sv2, sv3)
        dst_bufs = (dv0, dv1, dv2, dv3)
        idxg_bufs = (ix0, ix1)
        msg_bufs = (mg0, mg1)
        ea_bufs = (eb0, eb1)
        idx_sems = (isem0, isem1)
        gat_sems = (gsem0, gsem1)
        sct_sems = (ssem0, ssem1)

        def valid(u):
            return jnp.logical_and(u >= 0, s + u * NS < NCHUNKS)

        def zrow(r, carry):
            for k in range(HALF // 16):
                mg0[r, pl.ds(k * 16, 16)] = jnp.zeros((16,), jnp.float32)
            return carry
        lax.fori_loop(0, CH, zrow, 0, unroll=2)
        nblk = jnp.where(s < NS - 1, 8, 5)

        def zblk(t, carry):
            r0 = s * (8 * WB) + t * WB
            pltpu.sync_copy(mg0.at[pl.ds(0, WB)],
                            agg_sh.at[pl.ds(r0, WB)])
            return carry
        lax.fori_loop(0, nblk, zblk, 0)
        plsc.subcore_barrier()

        def group(t, carry):
            u0 = t * GRP

            idesc = []
            gdesc = []
            sdesc = []
            for i in range(GRP):
                base = (s + (u0 + i) * NS) * CH
                idesc.append((
                    pltpu.make_async_copy(srcg.at[pl.ds(base, CH)],
                                          src_bufs[i % 4], idx_sems[i % 2]),
                    pltpu.make_async_copy(dstg.at[pl.ds(base, CH)],
                                          dst_bufs[i % 4], idx_sems[i % 2]),
                ))
                gdesc.append((
                    pltpu.make_async_copy(h2.at[idxg_bufs[i % 2]],
                                          msg_bufs[i % 2], gat_sems[i % 2]),
                    pltpu.make_async_copy(ea2.at[pl.ds(c * EPAD + base, CH)],
                                          ea_bufs[i % 2], gat_sems[i % 2]),
                ))
                sdesc.append(
                    pltpu.make_async_copy(msg_bufs[i % 2],
                                          agg_sh.at[dst_bufs[i % 4]],
                                          sct_sems[i % 2]))

            def fire_idx_i(i):
                @pl.when(valid(u0 + i))
                def _(i=i):
                    idesc[i][0].start()
                    idesc[i][1].start()

            def proc(i):
                @pl.when(valid(u0 + i))
                def _(i=i):
                    gdesc[i][0].wait()
                    gdesc[i][1].wait()
                    msg_v, ea_v = msg_bufs[i % 2], ea_bufs[i % 2]

                    def crow(r, carry2):
                        for k in range(HALF // 16):
                            sl = pl.ds(k * 16, 16)
                            msg_v[r, sl] = jnp.maximum(
                                msg_v[r, sl] + ea_v[r, sl], 0.0)
                        return carry2
                    lax.fori_loop(0, CH, crow, 0, unroll=2)
                    sdesc[i].start(add=True)

            fire_idx_i(0)
            fire_idx_i(1)
            for i in range(GRP):
                u = u0 + i

                @pl.when(valid(u))
                def _(i=i, u=u):
                    idesc[i][0].wait()
                    idesc[i][1].wait()
                    src_v = src_bufs[i % 4]
                    idxg_v = idxg_bufs[i % 2]
                    for k in range(CH // 16):
                        sl = pl.ds(k * 16, 16)
                        idxg_v[sl] = src_v[sl] * 2 + c
                if i >= 2:
                    @pl.when(valid(u0 + i - 2))
                    def _(i=i):
                        sdesc[i - 2].wait()

                @pl.when(valid(u))
                def _(i=i):
                    gdesc[i][0].start()
                    gdesc[i][1].start()
                if i + 2 < GRP:
                    fire_idx_i(i + 2)
                if i >= 1:
                    proc(i - 1)
            proc(GRP - 1)

            @pl.when(valid(u0 + GRP - 2))
            def _():
                sdesc[GRP - 2].wait()

            @pl.when(valid(u0 + GRP - 1))
            def _():
                sdesc[GRP - 1].wait()
            return carry
        lax.fori_loop(0, NT // GRP, group, 0)

        plsc.subcore_barrier()

        def wblk(t, carry):
            r0 = s * (8 * WB) + t * WB
            pltpu.sync_copy(agg_sh.at[pl.ds(r0, WB)], out.at[c, pl.ds(r0, WB)])
            return carry
        lax.fori_loop(0, nblk, wblk, 0)

    return edge_kernel


_edge_kernel = _make_edge_kernel()


BR = 1000
G = N // BR


def _mlp1_body(h_ref, a0_ref, a1_ref, w1_ref, b1_ref, z_ref, s_ref):
    y = h_ref[...] + jnp.concatenate([a0_ref[...], a1_ref[...]], axis=1)
    z = lax.dot_general(y, w1_ref[...], (((1,), (1,)), ((), ())),
                        preferred_element_type=jnp.float32) + b1_ref[...]
    z_ref[...] = z
    s_ref[0, 0, :] = jnp.sum(z, axis=0)
    s_ref[0, 1, :] = jnp.sum(z * z, axis=0)


def _mlp1(h, a0, a1, w1, b1):
    return pl.pallas_call(
        _mlp1_body,
        grid=(G,),
        in_specs=[
            pl.BlockSpec((BR, DIM), lambda b: (b, 0)),
            pl.BlockSpec((BR, HALF), lambda b: (b, 0)),
            pl.BlockSpec((BR, HALF), lambda b: (b, 0)),
            pl.BlockSpec((DIM, DIM), lambda b: (0, 0)),
            pl.BlockSpec((1, DIM), lambda b: (0, 0)),
        ],
        out_specs=[
            pl.BlockSpec((BR, DIM), lambda b: (b, 0)),
            pl.BlockSpec((1, 2, DIM), lambda b: (b, 0, 0)),
        ],
        out_shape=[
            jax.ShapeDtypeStruct((N, DIM), jnp.float32),
            jax.ShapeDtypeStruct((G, 2, DIM), jnp.float32),
        ],
    )(h, a0, a1, w1, b1)


def _mlp2_body(z_ref, s_ref, g_ref, be_ref, w2_ref, b2_ref, o_ref):
    srt = s_ref[...]
    mu = jnp.sum(srt[:, 0, :], axis=0) * (1.0 / N)
    msq = jnp.sum(srt[:, 1, :], axis=0) * (1.0 / N)
    var = msq - mu * mu
    inv = lax.rsqrt(var + 1e-5)
    u = (z_ref[...] - mu) * (inv * g_ref[0]) + be_ref[0]
    u = u * jax.nn.sigmoid(u)
    v = lax.dot_general(u, w2_ref[...], (((1,), (1,)), ((), ())),
                        preferred_element_type=jnp.float32) + b2_ref[...]
    o_ref[...] = v * jax.nn.sigmoid(v)


def _mlp2(z, sums, gamma, beta, w2, b2):
    return pl.pallas_call(
        _mlp2_body,
        grid=(G,),
        in_specs=[
            pl.BlockSpec((BR, DIM), lambda b: (b, 0)),
            pl.BlockSpec((G, 2, DIM), lambda b: (0, 0, 0)),
            pl.BlockSpec((1, DIM), lambda b: (0, 0)),
            pl.BlockSpec((1, DIM), lambda b: (0, 0)),
            pl.BlockSpec((DIM, DIM), lambda b: (0, 0)),
            pl.BlockSpec((1, DIM), lambda b: (0, 0)),
        ],
        out_specs=pl.BlockSpec((BR, DIM), lambda b: (b, 0)),
        out_shape=jax.ShapeDtypeStruct((N, DIM), jnp.float32),
    )(z, sums, gamma, beta, w2, b2)


def kernel(x, edge_index, edge_attr, W1, b1, gamma, beta, W2, b2):
    src = jnp.concatenate([edge_index[0],
                           jnp.zeros((EPAD - E,), jnp.int32)])
    dst = jnp.concatenate([edge_index[1],
                           jnp.full((EPAD - E,), N, jnp.int32)])
    ea_p = jnp.concatenate([edge_attr,
                            jnp.zeros((EPAD - E, DIM), jnp.float32)])
    ea2 = ea_p.reshape(EPAD, 2, HALF).transpose(1, 0, 2).reshape(
        2 * EPAD, HALF)
    h = x
    for i in range(3):
        h2 = h.reshape(2 * N, HALF)
        aggs = _edge_kernel(h2, src, dst, ea2)
        z, sums = _mlp1(h, aggs[0], aggs[1], W1[i],
                        b1[i].reshape(1, DIM))
        h = _mlp2(z, sums, gamma[i].reshape(1, DIM), beta[i].reshape(1, DIM),
                  W2[i], b2[i].reshape(1, DIM))
    return h


# speedup vs baseline: 4.1500x; 2.0115x over previous
import functools

import jax
import jax.numpy as jnp
from jax import lax
from jax.experimental import pallas as pl
from jax.experimental.pallas import tpu as pltpu
from jax.experimental.pallas import tpu_sc as plsc

N = 10000
E = 160000
DIM = 256
HALF = 128
NC = 2
NS = 16
CH = 96
NCHUNKS = -(-E // CH)
EPAD = NCHUNKS * CH
NA = N + 8
WB = 80

_sc_mesh = plsc.VectorSubcoreMesh(core_axis_name="c", subcore_axis_name="s")


GRP = 16
NT = 112


def _make_edge_kernel():
    @functools.partial(
        pl.kernel,
        out_type=jax.ShapeDtypeStruct((NC, N, HALF), jnp.float32),
        mesh=_sc_mesh,
        scratch_types=[
            pltpu.VMEM((CH,), jnp.int32),
            pltpu.VMEM((CH,), jnp.int32),
            pltpu.VMEM((CH,), jnp.int32),
            pltpu.VMEM((CH,), jnp.int32),
            pltpu.VMEM((CH,), jnp.int32),
            pltpu.VMEM((CH,), jnp.int32),
            pltpu.VMEM((CH,), jnp.int32),
            pltpu.VMEM((CH,), jnp.int32),
            pltpu.VMEM((CH,), jnp.int32),
            pltpu.VMEM((CH,), jnp.int32),
            pltpu.VMEM((CH, HALF), jnp.float32),
            pltpu.VMEM((CH, HALF), jnp.float32),
            pltpu.VMEM((CH, HALF), jnp.float32),
            pltpu.VMEM((CH, HALF), jnp.float32),
            pltpu.VMEM_SHARED((NA, HALF), jnp.float32),
            pltpu.SemaphoreType.DMA,
            pltpu.SemaphoreType.DMA,
            pltpu.SemaphoreType.DMA,
            pltpu.SemaphoreType.DMA,
            pltpu.SemaphoreType.DMA,
            pltpu.SemaphoreType.DMA,
        ],
    )
    def edge_kernel(h2, srcg, dstg, ea2, out,
                    sv0, sv1, sv2, sv3, dv0, dv1, dv2, dv3,
                    ix0, ix1, mg0, mg1, eb0, eb1, agg_sh,
                    isem0, isem1, gsem0, gsem1, ssem0, ssem1):
        c = lax.axis_index("c")
        s = lax.axis_index("s")
        src_bufs = (sv0, sv1, sv2, sv3)
        dst_bufs = (dv0, dv1, dv2, dv3)
        idxg_bufs = (ix0, ix1)
        msg_bufs = (mg0, mg1)
        ea_bufs = (eb0, eb1)
        idx_sems = (isem0, isem1)
        gat_sems = (gsem0, gsem1)
        sct_sems = (ssem0, ssem1)

        def valid(u):
            return jnp.logical_and(u >= 0, s + u * NS < NCHUNKS)

        def zrow(r, carry):
            for k in range(HALF // 16):
                mg0[r, pl.ds(k * 16, 16)] = jnp.zeros((16,), jnp.float32)
            return carry
        lax.fori_loop(0, CH, zrow, 0, unroll=2)
        nblk = jnp.where(s < NS - 1, 8, 5)

        def zblk(t, carry):
            r0 = s * (8 * WB) + t * WB
            pltpu.sync_copy(mg0.at[pl.ds(0, WB)],
                            agg_sh.at[pl.ds(r0, WB)])
            return carry
        lax.fori_loop(0, nblk, zblk, 0)
        plsc.subcore_barrier()

        def group(t, carry):
            u0 = t * GRP

            idesc = []
            gdesc = []
            sdesc = []
            for i in range(GRP):
                base = (s + (u0 + i) * NS) * CH
                idesc.append((
                    pltpu.make_async_copy(srcg.at[pl.ds(base, CH)],
                                          src_bufs[i % 4], idx_sems[i % 2]),
                    pltpu.make_async_copy(dstg.at[pl.ds(base, CH)],
                                          dst_bufs[i % 4], idx_sems[i % 2]),
                ))
                gdesc.append((
                    pltpu.make_async_copy(h2.at[idxg_bufs[i % 2]],
                                          msg_bufs[i % 2], gat_sems[i % 2]),
                    pltpu.make_async_copy(ea2.at[pl.ds(c * EPAD + base, CH)],
                                          ea_bufs[i % 2], gat_sems[i % 2]),
                ))
                sdesc.append(
                    pltpu.make_async_copy(msg_bufs[i % 2],
                                          agg_sh.at[dst_bufs[i % 4]],
                                          sct_sems[i % 2]))

            def fire_idx_i(i):
                @pl.when(valid(u0 + i))
                def _(i=i):
                    idesc[i][0].start()
                    idesc[i][1].start()

            def proc(i):
                @pl.when(valid(u0 + i))
                def _(i=i):
                    gdesc[i][0].wait()
                    gdesc[i][1].wait()
                    msg_v, ea_v = msg_bufs[i % 2], ea_bufs[i % 2]

                    def crow(r, carry2):
                        hs = [msg_v[r, pl.ds(k * 16, 16)]
                              for k in range(HALF // 16)]
                        es = [ea_v[r, pl.ds(k * 16, 16)]
                              for k in range(HALF // 16)]
                        for k in range(HALF // 16):
                            msg_v[r, pl.ds(k * 16, 16)] = jnp.maximum(
                                hs[k] + es[k], 0.0)
                        return carry2
                    lax.fori_loop(0, CH, crow, 0, unroll=2)
                    sdesc[i].start(add=True)

            fire_idx_i(0)
            fire_idx_i(1)
            for i in range(GRP):
                u = u0 + i

                @pl.when(valid(u))
                def _(i=i, u=u):
                    idesc[i][0].wait()
                    idesc[i][1].wait()
                    src_v = src_bufs[i % 4]
                    idxg_v = idxg_bufs[i % 2]
                    for k in range(CH // 16):
                        sl = pl.ds(k * 16, 16)
                        idxg_v[sl] = src_v[sl] * 2 + c
                if i >= 2:
                    @pl.when(valid(u0 + i - 2))
                    def _(i=i):
                        sdesc[i - 2].wait()

                @pl.when(valid(u))
                def _(i=i):
                    gdesc[i][0].start()
                    gdesc[i][1].start()
                if i + 2 < GRP:
                    fire_idx_i(i + 2)
                if i >= 1:
                    proc(i - 1)
            proc(GRP - 1)

            @pl.when(valid(u0 + GRP - 2))
            def _():
                sdesc[GRP - 2].wait()

            @pl.when(valid(u0 + GRP - 1))
            def _():
                sdesc[GRP - 1].wait()
            return carry
        lax.fori_loop(0, NT // GRP, group, 0)

        plsc.subcore_barrier()

        def wblk(t, carry):
            r0 = s * (8 * WB) + t * WB
            pltpu.sync_copy(agg_sh.at[pl.ds(r0, WB)], out.at[c, pl.ds(r0, WB)])
            return carry
        lax.fori_loop(0, nblk, wblk, 0)

    return edge_kernel


_edge_kernel = _make_edge_kernel()


BR = 1000
G = N // BR


def _mlp1_body(h_ref, a0_ref, a1_ref, w1_ref, b1_ref, z_ref, s_ref):
    y = h_ref[...] + jnp.concatenate([a0_ref[...], a1_ref[...]], axis=1)
    z = lax.dot_general(y, w1_ref[...], (((1,), (1,)), ((), ())),
                        preferred_element_type=jnp.float32) + b1_ref[...]
    z_ref[...] = z
    s_ref[0, 0, :] = jnp.sum(z, axis=0)
    s_ref[0, 1, :] = jnp.sum(z * z, axis=0)


def _mlp1(h, a0, a1, w1, b1):
    return pl.pallas_call(
        _mlp1_body,
        grid=(G,),
        in_specs=[
            pl.BlockSpec((BR, DIM), lambda b: (b, 0)),
            pl.BlockSpec((BR, HALF), lambda b: (b, 0)),
            pl.BlockSpec((BR, HALF), lambda b: (b, 0)),
            pl.BlockSpec((DIM, DIM), lambda b: (0, 0)),
            pl.BlockSpec((1, DIM), lambda b: (0, 0)),
        ],
        out_specs=[
            pl.BlockSpec((BR, DIM), lambda b: (b, 0)),
            pl.BlockSpec((1, 2, DIM), lambda b: (b, 0, 0)),
        ],
        out_shape=[
            jax.ShapeDtypeStruct((N, DIM), jnp.float32),
            jax.ShapeDtypeStruct((G, 2, DIM), jnp.float32),
        ],
    )(h, a0, a1, w1, b1)


def _mlp2_body(z_ref, s_ref, g_ref, be_ref, w2_ref, b2_ref, o_ref):
    srt = s_ref[...]
    mu = jnp.sum(srt[:, 0, :], axis=0) * (1.0 / N)
    msq = jnp.sum(srt[:, 1, :], axis=0) * (1.0 / N)
    var = msq - mu * mu
    inv = lax.rsqrt(var + 1e-5)
    u = (z_ref[...] - mu) * (inv * g_ref[0]) + be_ref[0]
    u = u * jax.nn.sigmoid(u)
    v = lax.dot_general(u, w2_ref[...], (((1,), (1,)), ((), ())),
                        preferred_element_type=jnp.float32) + b2_ref[...]
    o_ref[...] = v * jax.nn.sigmoid(v)


def _mlp2(z, sums, gamma, beta, w2, b2):
    return pl.pallas_call(
        _mlp2_body,
        grid=(G,),
        in_specs=[
            pl.BlockSpec((BR, DIM), lambda b: (b, 0)),
            pl.BlockSpec((G, 2, DIM), lambda b: (0, 0, 0)),
            pl.BlockSpec((1, DIM), lambda b: (0, 0)),
            pl.BlockSpec((1, DIM), lambda b: (0, 0)),
            pl.BlockSpec((DIM, DIM), lambda b: (0, 0)),
            pl.BlockSpec((1, DIM), lambda b: (0, 0)),
        ],
        out_specs=pl.BlockSpec((BR, DIM), lambda b: (b, 0)),
        out_shape=jax.ShapeDtypeStruct((N, DIM), jnp.float32),
    )(z, sums, gamma, beta, w2, b2)


def kernel(x, edge_index, edge_attr, W1, b1, gamma, beta, W2, b2):
    src = jnp.concatenate([edge_index[0],
                           jnp.zeros((EPAD - E,), jnp.int32)])
    dst = jnp.concatenate([edge_index[1],
                           jnp.full((EPAD - E,), N, jnp.int32)])
    ea_p = jnp.concatenate([edge_attr,
                            jnp.zeros((EPAD - E, DIM), jnp.float32)])
    ea2 = ea_p.reshape(EPAD, 2, HALF).transpose(1, 0, 2).reshape(
        2 * EPAD, HALF)
    h = x
    for i in range(3):
        h2 = h.reshape(2 * N, HALF)
        aggs = _edge_kernel(h2, src, dst, ea2)
        z, sums = _mlp1(h, aggs[0], aggs[1], W1[i],
                        b1[i].reshape(1, DIM))
        h = _mlp2(z, sums, gamma[i].reshape(1, DIM), beta[i].reshape(1, DIM),
                  W2[i], b2[i].reshape(1, DIM))
    return h


# speedup vs baseline: 4.2446x; 1.0228x over previous
import functools

import jax
import jax.numpy as jnp
from jax import lax
from jax.experimental import pallas as pl
from jax.experimental.pallas import tpu as pltpu
from jax.experimental.pallas import tpu_sc as plsc

N = 10000
E = 160000
DIM = 256
HALF = 128
NC = 2
NS = 16
CH = 96
NCHUNKS = -(-E // CH)
EPAD = NCHUNKS * CH
NA = N + 8
WB = 80

_sc_mesh = plsc.VectorSubcoreMesh(core_axis_name="c", subcore_axis_name="s")


GRP = 28
NT = 112


def _make_edge_kernel():
    @functools.partial(
        pl.kernel,
        out_type=jax.ShapeDtypeStruct((NC, N, HALF), jnp.float32),
        mesh=_sc_mesh,
        scratch_types=[
            pltpu.VMEM((CH,), jnp.int32),
            pltpu.VMEM((CH,), jnp.int32),
            pltpu.VMEM((CH,), jnp.int32),
            pltpu.VMEM((CH,), jnp.int32),
            pltpu.VMEM((CH,), jnp.int32),
            pltpu.VMEM((CH,), jnp.int32),
            pltpu.VMEM((CH,), jnp.int32),
            pltpu.VMEM((CH,), jnp.int32),
            pltpu.VMEM((CH,), jnp.int32),
            pltpu.VMEM((CH,), jnp.int32),
            pltpu.VMEM((CH, HALF), jnp.float32),
            pltpu.VMEM((CH, HALF), jnp.float32),
            pltpu.VMEM((CH, HALF), jnp.float32),
            pltpu.VMEM((CH, HALF), jnp.float32),
            pltpu.VMEM_SHARED((NA, HALF), jnp.float32),
            pltpu.SemaphoreType.DMA,
            pltpu.SemaphoreType.DMA,
            pltpu.SemaphoreType.DMA,
            pltpu.SemaphoreType.DMA,
            pltpu.SemaphoreType.DMA,
            pltpu.SemaphoreType.DMA,
        ],
    )
    def edge_kernel(h2, srcg, dstg, ea2, out,
                    sv0, sv1, sv2, sv3, dv0, dv1, dv2, dv3,
                    ix0, ix1, mg0, mg1, eb0, eb1, agg_sh,
                    isem0, isem1, gsem0, gsem1, ssem0, ssem1):
        c = lax.axis_index("c")
        s = lax.axis_index("s")
        src_bufs = (sv0, sv1, sv2, sv3)
        dst_bufs = (dv0, dv1, dv2, dv3)
        idxg_bufs = (ix0, ix1)
        msg_bufs = (mg0, mg1)
        ea_bufs = (eb0, eb1)
        idx_sems = (isem0, isem1)
        gat_sems = (gsem0, gsem1)
        sct_sems = (ssem0, ssem1)

        def valid(u):
            return jnp.logical_and(u >= 0, s + u * NS < NCHUNKS)

        def zrow(r, carry):
            for k in range(HALF // 16):
                mg0[r, pl.ds(k * 16, 16)] = jnp.zeros((16,), jnp.float32)
            return carry
        lax.fori_loop(0, CH, zrow, 0, unroll=2)
        nblk = jnp.where(s < NS - 1, 8, 5)

        def zblk(t, carry):
            r0 = s * (8 * WB) + t * WB
            pltpu.sync_copy(mg0.at[pl.ds(0, WB)],
                            agg_sh.at[pl.ds(r0, WB)])
            return carry
        lax.fori_loop(0, nblk, zblk, 0)
        plsc.subcore_barrier()

        def group(t, carry):
            u0 = t * GRP

            idesc = []
            gdesc = []
            sdesc = []
            for i in range(GRP):
                base = (s + (u0 + i) * NS) * CH
                idesc.append((
                    pltpu.make_async_copy(srcg.at[pl.ds(base, CH)],
                                          src_bufs[i % 4], idx_sems[i % 2]),
                    pltpu.make_async_copy(dstg.at[pl.ds(base, CH)],
                                          dst_bufs[i % 4], idx_sems[i % 2]),
                ))
                gdesc.append((
                    pltpu.make_async_copy(h2.at[idxg_bufs[i % 2]],
                                          msg_bufs[i % 2], gat_sems[i % 2]),
                    pltpu.make_async_copy(ea2.at[pl.ds(c * EPAD + base, CH)],
                                          ea_bufs[i % 2], gat_sems[i % 2]),
                ))
                sdesc.append(
                    pltpu.make_async_copy(msg_bufs[i % 2],
                                          agg_sh.at[dst_bufs[i % 4]],
                                          sct_sems[i % 2]))

            def fire_idx_i(i):
                @pl.when(valid(u0 + i))
                def _(i=i):
                    idesc[i][0].start()
                    idesc[i][1].start()

            def proc(i):
                @pl.when(valid(u0 + i))
                def _(i=i):
                    gdesc[i][0].wait()
                    gdesc[i][1].wait()
                    msg_v, ea_v = msg_bufs[i % 2], ea_bufs[i % 2]

                    def crow(r, carry2):
                        hs = [msg_v[r, pl.ds(k * 16, 16)]
                              for k in range(HALF // 16)]
                        es = [ea_v[r, pl.ds(k * 16, 16)]
                              for k in range(HALF // 16)]
                        for k in range(HALF // 16):
                            msg_v[r, pl.ds(k * 16, 16)] = jnp.maximum(
                                hs[k] + es[k], 0.0)
                        return carry2
                    lax.fori_loop(0, CH, crow, 0, unroll=2)
                    sdesc[i].start(add=True)

            fire_idx_i(0)
            fire_idx_i(1)
            for i in range(GRP):
                u = u0 + i

                @pl.when(valid(u))
                def _(i=i, u=u):
                    idesc[i][0].wait()
                    idesc[i][1].wait()
                    src_v = src_bufs[i % 4]
                    idxg_v = idxg_bufs[i % 2]
                    for k in range(CH // 16):
                        sl = pl.ds(k * 16, 16)
                        idxg_v[sl] = src_v[sl] * 2 + c
                if i >= 2:
                    @pl.when(valid(u0 + i - 2))
                    def _(i=i):
                        sdesc[i - 2].wait()

                @pl.when(valid(u))
                def _(i=i):
                    gdesc[i][0].start()
                    gdesc[i][1].start()
                if i + 2 < GRP:
                    fire_idx_i(i + 2)
                if i >= 1:
                    proc(i - 1)
            proc(GRP - 1)

            @pl.when(valid(u0 + GRP - 2))
            def _():
                sdesc[GRP - 2].wait()

            @pl.when(valid(u0 + GRP - 1))
            def _():
                sdesc[GRP - 1].wait()
            return carry
        lax.fori_loop(0, NT // GRP, group, 0)

        plsc.subcore_barrier()

        def wblk(t, carry):
            r0 = s * (8 * WB) + t * WB
            pltpu.sync_copy(agg_sh.at[pl.ds(r0, WB)], out.at[c, pl.ds(r0, WB)])
            return carry
        lax.fori_loop(0, nblk, wblk, 0)

    return edge_kernel


_edge_kernel = _make_edge_kernel()


BR = 1000
G = N // BR


def _mlp1_body(h_ref, a0_ref, a1_ref, w1_ref, b1_ref, z_ref, s_ref):
    y = h_ref[...] + jnp.concatenate([a0_ref[...], a1_ref[...]], axis=1)
    z = lax.dot_general(y, w1_ref[...], (((1,), (1,)), ((), ())),
                        preferred_element_type=jnp.float32) + b1_ref[...]
    z_ref[...] = z
    s_ref[0, 0, :] = jnp.sum(z, axis=0)
    s_ref[0, 1, :] = jnp.sum(z * z, axis=0)


def _mlp1(h, a0, a1, w1, b1):
    return pl.pallas_call(
        _mlp1_body,
        grid=(G,),
        in_specs=[
            pl.BlockSpec((BR, DIM), lambda b: (b, 0)),
            pl.BlockSpec((BR, HALF), lambda b: (b, 0)),
            pl.BlockSpec((BR, HALF), lambda b: (b, 0)),
            pl.BlockSpec((DIM, DIM), lambda b: (0, 0)),
            pl.BlockSpec((1, DIM), lambda b: (0, 0)),
        ],
        out_specs=[
            pl.BlockSpec((BR, DIM), lambda b: (b, 0)),
            pl.BlockSpec((1, 2, DIM), lambda b: (b, 0, 0)),
        ],
        out_shape=[
            jax.ShapeDtypeStruct((N, DIM), jnp.float32),
            jax.ShapeDtypeStruct((G, 2, DIM), jnp.float32),
        ],
    )(h, a0, a1, w1, b1)


def _mlp2_body(z_ref, s_ref, g_ref, be_ref, w2_ref, b2_ref, o_ref):
    srt = s_ref[...]
    mu = jnp.sum(srt[:, 0, :], axis=0) * (1.0 / N)
    msq = jnp.sum(srt[:, 1, :], axis=0) * (1.0 / N)
    var = msq - mu * mu
    inv = lax.rsqrt(var + 1e-5)
    u = (z_ref[...] - mu) * (inv * g_ref[0]) + be_ref[0]
    u = u * jax.nn.sigmoid(u)
    v = lax.dot_general(u, w2_ref[...], (((1,), (1,)), ((), ())),
                        preferred_element_type=jnp.float32) + b2_ref[...]
    o_ref[...] = v * jax.nn.sigmoid(v)


def _mlp2(z, sums, gamma, beta, w2, b2):
    return pl.pallas_call(
        _mlp2_body,
        grid=(G,),
        in_specs=[
            pl.BlockSpec((BR, DIM), lambda b: (b, 0)),
            pl.BlockSpec((G, 2, DIM), lambda b: (0, 0, 0)),
            pl.BlockSpec((1, DIM), lambda b: (0, 0)),
            pl.BlockSpec((1, DIM), lambda b: (0, 0)),
            pl.BlockSpec((DIM, DIM), lambda b: (0, 0)),
            pl.BlockSpec((1, DIM), lambda b: (0, 0)),
        ],
        out_specs=pl.BlockSpec((BR, DIM), lambda b: (b, 0)),
        out_shape=jax.ShapeDtypeStruct((N, DIM), jnp.float32),
    )(z, sums, gamma, beta, w2, b2)


def kernel(x, edge_index, edge_attr, W1, b1, gamma, beta, W2, b2):
    src = jnp.concatenate([edge_index[0],
                           jnp.zeros((EPAD - E,), jnp.int32)])
    dst = jnp.concatenate([edge_index[1],
                           jnp.full((EPAD - E,), N, jnp.int32)])
    ea_p = jnp.concatenate([edge_attr,
                            jnp.zeros((EPAD - E, DIM), jnp.float32)])
    ea2 = ea_p.reshape(EPAD, 2, HALF).transpose(1, 0, 2).reshape(
        2 * EPAD, HALF)
    h = x
    for i in range(3):
        h2 = h.reshape(2 * N, HALF)
        aggs = _edge_kernel(h2, src, dst, ea2)
        z, sums = _mlp1(h, aggs[0], aggs[1], W1[i],
                        b1[i].reshape(1, DIM))
        h = _mlp2(z, sums, gamma[i].reshape(1, DIM), beta[i].reshape(1, DIM),
                  W2[i], b2[i].reshape(1, DIM))
    return h


# speedup vs baseline: 4.2662x; 1.0051x over previous
import functools

import jax
import jax.numpy as jnp
from jax import lax
from jax.experimental import pallas as pl
from jax.experimental.pallas import tpu as pltpu
from jax.experimental.pallas import tpu_sc as plsc

N = 10000
E = 160000
DIM = 256
HALF = 128
NC = 2
NS = 16
CH = 96
NCHUNKS = -(-E // CH)
EPAD = NCHUNKS * CH
NA = N + 8
WB = 80

_sc_mesh = plsc.VectorSubcoreMesh(core_axis_name="c", subcore_axis_name="s")


GRP = 28
NT = 112


def _make_edge_kernel():
    @functools.partial(
        pl.kernel,
        out_type=jax.ShapeDtypeStruct((NC, N, HALF), jnp.float32),
        mesh=_sc_mesh,
        scratch_types=[
            pltpu.VMEM((CH,), jnp.int32),
            pltpu.VMEM((CH,), jnp.int32),
            pltpu.VMEM((CH,), jnp.int32),
            pltpu.VMEM((CH,), jnp.int32),
            pltpu.VMEM((CH,), jnp.int32),
            pltpu.VMEM((CH,), jnp.int32),
            pltpu.VMEM((CH,), jnp.int32),
            pltpu.VMEM((CH,), jnp.int32),
            pltpu.VMEM((CH,), jnp.int32),
            pltpu.VMEM((CH,), jnp.int32),
            pltpu.VMEM((CH, HALF), jnp.float32),
            pltpu.VMEM((CH, HALF), jnp.float32),
            pltpu.VMEM((CH, HALF), jnp.float32),
            pltpu.VMEM((CH, HALF), jnp.float32),
            pltpu.VMEM_SHARED((NA, HALF), jnp.float32),
            pltpu.SemaphoreType.DMA,
            pltpu.SemaphoreType.DMA,
            pltpu.SemaphoreType.DMA,
            pltpu.SemaphoreType.DMA,
            pltpu.SemaphoreType.DMA,
            pltpu.SemaphoreType.DMA,
        ],
    )
    def edge_kernel(h2, srcg, dstg, ea2, out,
                    sv0, sv1, sv2, sv3, dv0, dv1, dv2, dv3,
                    ix0, ix1, mg0, mg1, eb0, eb1, agg_sh,
                    isem0, isem1, gsem0, gsem1, ssem0, ssem1):
        c = lax.axis_index("c")
        s = lax.axis_index("s")
        src_bufs = (sv0, sv1, sv2, sv3)
        dst_bufs = (dv0, dv1, dv2, dv3)
        idxg_bufs = (ix0, ix1)
        msg_bufs = (mg0, mg1)
        ea_bufs = (eb0, eb1)
        idx_sems = (isem0, isem1)
        gat_sems = (gsem0, gsem1)
        sct_sems = (ssem0, ssem1)

        def valid(u):
            return jnp.logical_and(u >= 0, s + u * NS < NCHUNKS)

        def zrow(r, carry):
            for k in range(HALF // 16):
                mg0[r, pl.ds(k * 16, 16)] = jnp.zeros((16,), jnp.float32)
            return carry
        lax.fori_loop(0, CH, zrow, 0, unroll=2)
        nblk = jnp.where(s < NS - 1, 8, 5)

        def zblk(t, carry):
            r0 = s * (8 * WB) + t * WB
            pltpu.sync_copy(mg0.at[pl.ds(0, WB)],
                            agg_sh.at[pl.ds(r0, WB)])
            return carry
        lax.fori_loop(0, nblk, zblk, 0)
        plsc.subcore_barrier()

        def group(t, carry):
            u0 = t * GRP

            idesc = []
            gdesc = []
            sdesc = []
            for i in range(GRP):
                base = (s + (u0 + i) * NS) * CH
                idesc.append((
                    pltpu.make_async_copy(srcg.at[pl.ds(base, CH)],
                                          src_bufs[i % 4], idx_sems[i % 2]),
                    pltpu.make_async_copy(dstg.at[pl.ds(base, CH)],
                                          dst_bufs[i % 4], idx_sems[i % 2]),
                ))
                gdesc.append((
                    pltpu.make_async_copy(h2.at[idxg_bufs[i % 2]],
                                          msg_bufs[i % 2], gat_sems[i % 2]),
                    pltpu.make_async_copy(ea2.at[pl.ds(c * EPAD + base, CH)],
                                          ea_bufs[i % 2], gat_sems[i % 2]),
                ))
                sdesc.append(
                    pltpu.make_async_copy(msg_bufs[i % 2],
                                          agg_sh.at[dst_bufs[i % 4]],
                                          sct_sems[i % 2]))

            def fire_idx_i(i):
                @pl.when(valid(u0 + i))
                def _(i=i):
                    idesc[i][0].start()
                    idesc[i][1].start()

            def proc(i):
                @pl.when(valid(u0 + i))
                def _(i=i):
                    gdesc[i][0].wait()
                    gdesc[i][1].wait()
                    msg_v, ea_v = msg_bufs[i % 2], ea_bufs[i % 2]

                    def crow(r, carry2):
                        hs = [msg_v[r, pl.ds(k * 16, 16)]
                              for k in range(HALF // 16)]
                        es = [ea_v[r, pl.ds(k * 16, 16)]
                              for k in range(HALF // 16)]
                        for k in range(HALF // 16):
                            msg_v[r, pl.ds(k * 16, 16)] = jnp.maximum(
                                hs[k] + es[k], 0.0)
                        return carry2
                    lax.fori_loop(0, CH, crow, 0, unroll=4)
                    sdesc[i].start(add=True)

            fire_idx_i(0)
            fire_idx_i(1)
            for i in range(GRP):
                u = u0 + i

                @pl.when(valid(u))
                def _(i=i, u=u):
                    idesc[i][0].wait()
                    idesc[i][1].wait()
                    src_v = src_bufs[i % 4]
                    idxg_v = idxg_bufs[i % 2]
                    for k in range(CH // 16):
                        sl = pl.ds(k * 16, 16)
                        idxg_v[sl] = src_v[sl] * 2 + c
                if i >= 2:
                    @pl.when(valid(u0 + i - 2))
                    def _(i=i):
                        sdesc[i - 2].wait()

                @pl.when(valid(u))
                def _(i=i):
                    gdesc[i][0].start()
                    gdesc[i][1].start()
                if i + 2 < GRP:
                    fire_idx_i(i + 2)
                if i >= 1:
                    proc(i - 1)
            proc(GRP - 1)

            @pl.when(valid(u0 + GRP - 2))
            def _():
                sdesc[GRP - 2].wait()

            @pl.when(valid(u0 + GRP - 1))
            def _():
                sdesc[GRP - 1].wait()
            return carry
        lax.fori_loop(0, NT // GRP, group, 0)

        plsc.subcore_barrier()

        def wblk(t, carry):
            r0 = s * (8 * WB) + t * WB
            pltpu.sync_copy(agg_sh.at[pl.ds(r0, WB)], out.at[c, pl.ds(r0, WB)])
            return carry
        lax.fori_loop(0, nblk, wblk, 0)

    return edge_kernel


_edge_kernel = _make_edge_kernel()


BR = 2000
G = N // BR


def _mlp1_body(h_ref, a0_ref, a1_ref, w1_ref, b1_ref, z_ref, s_ref):
    y = h_ref[...] + jnp.concatenate([a0_ref[...], a1_ref[...]], axis=1)
    z = lax.dot_general(y, w1_ref[...], (((1,), (1,)), ((), ())),
                        preferred_element_type=jnp.float32) + b1_ref[...]
    z_ref[...] = z
    s_ref[0, 0, :] = jnp.sum(z, axis=0)
    s_ref[0, 1, :] = jnp.sum(z * z, axis=0)


def _mlp1(h, a0, a1, w1, b1):
    return pl.pallas_call(
        _mlp1_body,
        grid=(G,),
        in_specs=[
            pl.BlockSpec((BR, DIM), lambda b: (b, 0)),
            pl.BlockSpec((BR, HALF), lambda b: (b, 0)),
            pl.BlockSpec((BR, HALF), lambda b: (b, 0)),
            pl.BlockSpec((DIM, DIM), lambda b: (0, 0)),
            pl.BlockSpec((1, DIM), lambda b: (0, 0)),
        ],
        out_specs=[
            pl.BlockSpec((BR, DIM), lambda b: (b, 0)),
            pl.BlockSpec((1, 2, DIM), lambda b: (b, 0, 0)),
        ],
        out_shape=[
            jax.ShapeDtypeStruct((N, DIM), jnp.float32),
            jax.ShapeDtypeStruct((G, 2, DIM), jnp.float32),
        ],
    )(h, a0, a1, w1, b1)


def _mlp2_body(z_ref, s_ref, g_ref, be_ref, w2_ref, b2_ref, o_ref):
    srt = s_ref[...]
    mu = jnp.sum(srt[:, 0, :], axis=0) * (1.0 / N)
    msq = jnp.sum(srt[:, 1, :], axis=0) * (1.0 / N)
    var = msq - mu * mu
    inv = lax.rsqrt(var + 1e-5)
    u = (z_ref[...] - mu) * (inv * g_ref[0]) + be_ref[0]
    u = u * jax.nn.sigmoid(u)
    v = lax.dot_general(u, w2_ref[...], (((1,), (1,)), ((), ())),
                        preferred_element_type=jnp.float32) + b2_ref[...]
    o_ref[...] = v * jax.nn.sigmoid(v)


def _mlp2(z, sums, gamma, beta, w2, b2):
    return pl.pallas_call(
        _mlp2_body,
        grid=(G,),
        in_specs=[
            pl.BlockSpec((BR, DIM), lambda b: (b, 0)),
            pl.BlockSpec((G, 2, DIM), lambda b: (0, 0, 0)),
            pl.BlockSpec((1, DIM), lambda b: (0, 0)),
            pl.BlockSpec((1, DIM), lambda b: (0, 0)),
            pl.BlockSpec((DIM, DIM), lambda b: (0, 0)),
            pl.BlockSpec((1, DIM), lambda b: (0, 0)),
        ],
        out_specs=pl.BlockSpec((BR, DIM), lambda b: (b, 0)),
        out_shape=jax.ShapeDtypeStruct((N, DIM), jnp.float32),
    )(z, sums, gamma, beta, w2, b2)


def kernel(x, edge_index, edge_attr, W1, b1, gamma, beta, W2, b2):
    src = jnp.concatenate([edge_index[0],
                           jnp.zeros((EPAD - E,), jnp.int32)])
    dst = jnp.concatenate([edge_index[1],
                           jnp.full((EPAD - E,), N, jnp.int32)])
    ea_p = jnp.concatenate([edge_attr,
                            jnp.zeros((EPAD - E, DIM), jnp.float32)])
    ea2 = ea_p.reshape(EPAD, 2, HALF).transpose(1, 0, 2).reshape(
        2 * EPAD, HALF)
    h = x
    for i in range(3):
        h2 = h.reshape(2 * N, HALF)
        aggs = _edge_kernel(h2, src, dst, ea2)
        z, sums = _mlp1(h, aggs[0], aggs[1], W1[i],
                        b1[i].reshape(1, DIM))
        h = _mlp2(z, sums, gamma[i].reshape(1, DIM), beta[i].reshape(1, DIM),
                  W2[i], b2[i].reshape(1, DIM))
    return h
